# baseline (speedup 1.0000x reference)
import functools

import jax
import jax.numpy as jnp
from jax import lax
from jax.experimental import pallas as pl
from jax.experimental.pallas import tpu as pltpu
from jax.experimental.pallas import tpu_sc as plsc

N_ORDER = 10000
N_RIDER = 10000
E = 160000
D_ORDER = 256
D_RIDER = 128
D_EDGE = 16
H = 256
HH = 128
NT = 16
EPT = E // NT
CH = 80
NCH = EPT // CH
SCC = 5
CPS = NCH // SCC
NP = 10240
RB = 10
ROWS = N_ORDER // RB


def _order_dense_body(x_ref, ws_ref, wp_ref, att_ref, bp_ref,
                      hlo_ref, hhi_ref, plo_ref, phi_ref, a_ref, amax_ref):
    i = pl.program_id(0)
    x = x_ref[...]
    h = jnp.dot(x, ws_ref[...], preferred_element_type=jnp.float32)
    hlo_ref[...] = h[:, :HH]
    hhi_ref[...] = h[:, HH:]
    a = jnp.dot(h, att_ref[...], preferred_element_type=jnp.float32)
    a_ref[...] = a
    proj = jnp.dot(x, wp_ref[...], preferred_element_type=jnp.float32)
    proj = proj + bp_ref[...]
    plo_ref[...] = proj[:, :HH]
    phi_ref[...] = proj[:, HH:]
    m = jnp.max(a).reshape(1, 1)

    @pl.when(i == 0)
    def _():
        amax_ref[...] = m

    @pl.when(i != 0)
    def _():
        amax_ref[...] = jnp.maximum(amax_ref[...], m)


def _order_dense(x_order, W_src, W_proj, att_src, b_proj):
    att2 = att_src.reshape(H, 1)
    bp2 = b_proj.reshape(1, H)
    out_shapes = [
        jax.ShapeDtypeStruct((N_ORDER, HH), jnp.float32),
        jax.ShapeDtypeStruct((N_ORDER, HH), jnp.float32),
        jax.ShapeDtypeStruct((N_ORDER, HH), jnp.float32),
        jax.ShapeDtypeStruct((N_ORDER, HH), jnp.float32),
        jax.ShapeDtypeStruct((N_ORDER, 1), jnp.float32),
        jax.ShapeDtypeStruct((1, 1), jnp.float32),
    ]
    row_spec = pl.BlockSpec((ROWS, HH), lambda i: (i, 0))
    return pl.pallas_call(
        _order_dense_body,
        grid=(RB,),
        in_specs=[
            pl.BlockSpec((ROWS, D_ORDER), lambda i: (i, 0)),
            pl.BlockSpec((D_ORDER, H), lambda i: (0, 0)),
            pl.BlockSpec((D_ORDER, H), lambda i: (0, 0)),
            pl.BlockSpec((H, 1), lambda i: (0, 0)),
            pl.BlockSpec((1, H), lambda i: (0, 0)),
        ],
        out_specs=[
            row_spec, row_spec, row_spec, row_spec,
            pl.BlockSpec((ROWS, 1), lambda i: (i, 0)),
            pl.BlockSpec((1, 1), lambda i: (0, 0)),
        ],
        out_shape=out_shapes,
    )(x_order, W_src, W_proj, att2, bp2)


def _rider_dense_body(x_ref, wd_ref, att_ref, a_ref, amax_ref):
    i = pl.program_id(0)
    hd = jnp.dot(x_ref[...], wd_ref[...], preferred_element_type=jnp.float32)
    a = jnp.dot(hd, att_ref[...], preferred_element_type=jnp.float32)
    a_ref[...] = a
    m = jnp.max(a).reshape(1, 1)

    @pl.when(i == 0)
    def _():
        amax_ref[...] = m

    @pl.when(i != 0)
    def _():
        amax_ref[...] = jnp.maximum(amax_ref[...], m)


def _rider_dense(x_rider, W_dst, att_dst):
    att2 = att_dst.reshape(H, 1)
    return pl.pallas_call(
        _rider_dense_body,
        grid=(RB,),
        in_specs=[
            pl.BlockSpec((ROWS, D_RIDER), lambda i: (i, 0)),
            pl.BlockSpec((D_RIDER, H), lambda i: (0, 0)),
            pl.BlockSpec((H, 1), lambda i: (0, 0)),
        ],
        out_specs=[
            pl.BlockSpec((ROWS, 1), lambda i: (i, 0)),
            pl.BlockSpec((1, 1), lambda i: (0, 0)),
        ],
        out_shape=[
            jax.ShapeDtypeStruct((N_RIDER, 1), jnp.float32),
            jax.ShapeDtypeStruct((1, 1), jnp.float32),
        ],
    )(x_rider, W_dst, att2)


def _gate_body(a_ref, w1_ref, b1_ref, w2_ref, b2_ref, o_ref):
    g = jnp.dot(a_ref[...], w1_ref[...], preferred_element_type=jnp.float32)
    g = jnp.maximum(g + b1_ref[...], 0.0)
    s = jnp.dot(g, w2_ref[...], preferred_element_type=jnp.float32)
    o_ref[...] = jax.nn.sigmoid(s + b2_ref[...])


def _gate(edge_attr, Wg1, bg1, Wg2, bg2):
    A = edge_attr.reshape(E // 8, 128)
    eye8 = jnp.eye(8, dtype=jnp.float32)
    W1b = jnp.kron(eye8, Wg1)
    b1b = jnp.tile(bg1, 8).reshape(1, 128)
    W2b = jnp.kron(eye8, Wg2)
    b2b = jnp.tile(bg2, 8).reshape(1, 8)
    rows = (E // 8) // RB
    out = pl.pallas_call(
        _gate_body,
        grid=(RB,),
        in_specs=[
            pl.BlockSpec((rows, 128), lambda i: (i, 0)),
            pl.BlockSpec((128, 128), lambda i: (0, 0)),
            pl.BlockSpec((1, 128), lambda i: (0, 0)),
            pl.BlockSpec((128, 8), lambda i: (0, 0)),
            pl.BlockSpec((1, 8), lambda i: (0, 0)),
        ],
        out_specs=pl.BlockSpec((rows, 8), lambda i: (i, 0)),
        out_shape=jax.ShapeDtypeStruct((E // 8, 8), jnp.float32),
    )(A, W1b, b1b, W2b, b2b)
    return out.reshape(E)


def _combine_body(p0_ref, p1_ref, g_ref, o_ref):
    o_ref[...] = (p0_ref[...] + p1_ref[...]) * g_ref[...]


def _combine(p0, p1, gate):
    shp = (E // 128, 128)
    return pl.pallas_call(
        _combine_body,
        out_shape=jax.ShapeDtypeStruct(shp, jnp.float32),
    )(p0.reshape(shp), p1.reshape(shp), gate.reshape(shp)).reshape(E)


def _rload(ref, r, j):
    return ref[r, pl.ds(16 * j, 16)]


def _rstore(ref, r, j, v):
    ref[r, pl.ds(16 * j, 16)] = v


def _dsplit(d16):
    return [lax.shift_right_logical(d16, 7), jnp.bitwise_and(d16, 127)]


@functools.partial(
    pl.kernel,
    out_type=[
        jax.ShapeDtypeStruct((2, NT, SCC, CPS, CH), jnp.float32),
        jax.ShapeDtypeStruct((2, NT, SCC, CPS, CH), jnp.float32),
    ],
    mesh=plsc.VectorSubcoreMesh(core_axis_name="c", subcore_axis_name="s"),
    compiler_params=pltpu.CompilerParams(
        use_tc_tiling_on_sc=False, needs_layout_passes=False),
    scratch_types=[
        pltpu.VMEM((CPS, CH), jnp.int32),
        pltpu.VMEM((CPS, CH), jnp.int32),
        pltpu.VMEM((CPS, CH), jnp.float32),
        pltpu.VMEM((80, 128), jnp.float32),
        pltpu.VMEM((80, 128), jnp.float32),
        pltpu.VMEM((80, 128), jnp.float32),
        pltpu.VMEM((80, 128), jnp.float32),
        pltpu.VMEM((16,), jnp.float32),
        pltpu.VMEM((1, CH), jnp.int32),
        pltpu.SemaphoreType.DMA,
        pltpu.SemaphoreType.DMA,
        pltpu.VMEM_SHARED((N_RIDER, HH), jnp.float32),
        pltpu.VMEM_SHARED((80, 128), jnp.float32),
    ],
)
def _sc_edges(src4_h, dst4_h, asrc_h, adst_h, cvec_h, zrows_h, brows_h,
              hlo_h, hhi_h, plo_h, phi_h, out_h, exb_h,
              srcsc_v, dstsc_v, stagef_v, den_v, rowsA, rowsB, rowsC,
              cvec_v, idxr_v, sem0, sem1, rider_s, dfin_s):
    c = lax.axis_index("c")
    t = lax.axis_index("s")

    pltpu.sync_copy(asrc_h, rowsB)
    pltpu.sync_copy(adst_h, rowsA)
    pltpu.sync_copy(cvec_h, cvec_v)
    pltpu.sync_copy(zrows_h, den_v)
    pltpu.sync_copy(brows_h.at[c],
                    rider_s.at[pl.ds((N_RIDER // NT) * t, N_RIDER // NT)])

    @pl.when(t == 0)
    def _():
        pltpu.sync_copy(zrows_h, dfin_s)

    for u in range(CH // 16):
        idxr_v[0, pl.ds(16 * u, 16)] = lax.iota(jnp.int32, 16) + 16 * u

    cv = cvec_v[...]

    def superA(s, carry):
        pltpu.sync_copy(src4_h.at[t, s], srcsc_v)
        pltpu.sync_copy(dst4_h.at[t, s], dstsc_v)

        def rA(r, cc):
            for u in range(CH // 16):
                s16 = _rload(srcsc_v, r, u)
                d16 = _rload(dstsc_v, r, u)
                av = plsc.load_gather(rowsB, _dsplit(s16))
                bv = plsc.load_gather(rowsA, _dsplit(d16))
                x = av + bv
                ex = jnp.exp(jnp.where(x >= 0.0, x, 0.2 * x) - cv)
                _rstore(stagef_v, r, u, ex)
                plsc.addupdate_scatter(den_v, _dsplit(d16), ex)
            return cc

        lax.fori_loop(0, CPS, rA, 0)
        pltpu.sync_copy(stagef_v, exb_h.at[c, t, s])
        return carry

    lax.fori_loop(0, SCC, superA, 0)

    plsc.subcore_barrier()
    pltpu.sync_copy(den_v, dfin_s.at[idxr_v.at[0]], add=True)
    plsc.subcore_barrier()
    pltpu.sync_copy(dfin_s, den_v)

    def superN(s, carry):
        pltpu.sync_copy(dst4_h.at[t, s], dstsc_v)
        pltpu.sync_copy(exb_h.at[c, t, s], stagef_v)

        def rN(r, cc):
            for u in range(CH // 16):
                d16 = _rload(dstsc_v, r, u)
                dv = plsc.load_gather(den_v, _dsplit(d16))
                _rstore(stagef_v, r, u,
                        _rload(stagef_v, r, u) / (dv + 1e-16))
            return cc

        lax.fori_loop(0, CPS, rN, 0)
        pltpu.sync_copy(stagef_v, exb_h.at[c, t, s])
        return carry

    lax.fori_loop(0, SCC, superN, 0)

    def phase23(h_h, p_h):
        def scale_scatter(r, gbuf):
            for k in range(CH // 16):
                a16 = _rload(stagef_v, r, k)
                rid = lax.iota(jnp.int32, 16) + 16 * k

                def sj(j8, cc):
                    for jj in range(8):
                        colv = jnp.full((16,), j8 * 8 + jj, jnp.int32)
                        v = plsc.load_gather(gbuf, [rid, colv])
                        plsc.store_scatter(gbuf, [rid, colv], v * a16)
                    return cc

                lax.fori_loop(0, HH // 8, sj, 0)
            pltpu.sync_copy(gbuf, rider_s.at[dstsc_v.at[r]], add=True)

        def superB(s, carry):
            pltpu.sync_copy(src4_h.at[t, s], srcsc_v)
            pltpu.sync_copy(dst4_h.at[t, s], dstsc_v)
            pltpu.sync_copy(exb_h.at[c, t, s], stagef_v)
            pltpu.make_async_copy(h_h.at[srcsc_v.at[0]], rowsB, sem0).start()

            def rB(r, cc):
                nxt = r + 1

                @pl.when(jnp.logical_and(nxt < CPS, nxt % 2 == 1))
                def _():
                    pltpu.make_async_copy(
                        h_h.at[srcsc_v.at[nxt]], rowsC, sem1).start()

                @pl.when(jnp.logical_and(nxt < CPS, nxt % 2 == 0))
                def _():
                    pltpu.make_async_copy(
                        h_h.at[srcsc_v.at[nxt]], rowsB, sem0).start()

                @pl.when(r % 2 == 0)
                def _():
                    pltpu.make_async_copy(
                        h_h.at[srcsc_v.at[r]], rowsB, sem0).wait()
                    scale_scatter(r, rowsB)

                @pl.when(r % 2 == 1)
                def _():
                    pltpu.make_async_copy(
                        h_h.at[srcsc_v.at[r]], rowsC, sem1).wait()
                    scale_scatter(r, rowsC)

                return cc

            lax.fori_loop(0, CPS, rB, 0)
            return carry

        lax.fori_loop(0, SCC, superB, 0)
        plsc.subcore_barrier()

        def dot_rows(r, pbuf):
            pltpu.sync_copy(rider_s.at[dstsc_v.at[r]], rowsA)
            for k in range(CH // 16):
                rid = lax.iota(jnp.int32, 16) + 16 * k

                def dj(j8, acc):
                    for jj in range(8):
                        colv = jnp.full((16,), j8 * 8 + jj, jnp.int32)
                        pv = plsc.load_gather(pbuf, [rid, colv])
                        rv = plsc.load_gather(rowsA, [rid, colv])
                        acc = acc + pv * rv
                    return acc

                acc = lax.fori_loop(0, HH // 8, dj,
                                    jnp.zeros((16,), jnp.float32))
                _rstore(stagef_v, r, k, acc)

        def superC(s, carry):
            pltpu.sync_copy(src4_h.at[t, s], srcsc_v)
            pltpu.sync_copy(dst4_h.at[t, s], dstsc_v)
            pltpu.make_async_copy(p_h.at[srcsc_v.at[0]], rowsB, sem0).start()

            def rC(r, cc):
                nxt = r + 1

                @pl.when(jnp.logical_and(nxt < CPS, nxt % 2 == 1))
                def _():
                    pltpu.make_async_copy(
                        p_h.at[srcsc_v.at[nxt]], rowsC, sem1).start()

                @pl.when(jnp.logical_and(nxt < CPS, nxt % 2 == 0))
                def _():
                    pltpu.make_async_copy(
                        p_h.at[srcsc_v.at[nxt]], rowsB, sem0).start()

                @pl.when(r % 2 == 0)
                def _():
                    pltpu.make_async_copy(
                        p_h.at[srcsc_v.at[r]], rowsB, sem0).wait()
                    dot_rows(r, rowsB)

                @pl.when(r % 2 == 1)
                def _():
                    pltpu.make_async_copy(
                        p_h.at[srcsc_v.at[r]], rowsC, sem1).wait()
                    dot_rows(r, rowsC)

                return cc

            lax.fori_loop(0, CPS, rC, 0)
            pltpu.sync_copy(stagef_v, out_h.at[c, t, s])
            return carry

        lax.fori_loop(0, SCC, superC, 0)

    @pl.when(c == 0)
    def _():
        phase23(hlo_h, plo_h)

    @pl.when(c == 1)
    def _():
        phase23(hhi_h, phi_h)


def kernel(x_order, x_rider, edge_index, edge_attr, W_src, W_dst,
           att_src, att_dst, bias_gat, W_proj, b_proj, Wg1, bg1, Wg2, bg2):
    src = edge_index[0].astype(jnp.int32)
    dst = edge_index[1].astype(jnp.int32)

    h_lo, h_hi, p_lo, p_hi, a_src, amax_s = _order_dense(
        x_order, W_src, W_proj, att_src, b_proj)
    a_dst, amax_d = _rider_dense(x_rider, W_dst, att_dst)
    gate = _gate(edge_attr, Wg1, bg1, Wg2, bg2)

    c = jnp.maximum(amax_s[0, 0] + amax_d[0, 0], 0.0)
    cvec = jnp.broadcast_to(c, (16,))
    pad = jnp.zeros((NP - N_ORDER,), jnp.float32)
    asrc2d = jnp.concatenate([a_src.reshape(N_ORDER), pad]).reshape(80, 128)
    adst2d = jnp.concatenate([a_dst.reshape(N_RIDER), pad]).reshape(80, 128)
    zrows = jnp.zeros((80, 128), jnp.float32)
    brows = jnp.broadcast_to(bias_gat.reshape(2, 1, HH),
                             (2, N_RIDER // NT, HH))

    partial, _ = _sc_edges(
        src.reshape(NT, SCC, CPS, CH), dst.reshape(NT, SCC, CPS, CH),
        asrc2d, adst2d, cvec, zrows, brows, h_lo, h_hi, p_lo, p_hi)

    p = partial.reshape(2, E)
    return _combine(p[0], p[1], gate)


# speedup vs baseline: 7.4072x; 7.4072x over previous
import functools

import jax
import jax.numpy as jnp
from jax import lax
from jax.experimental import pallas as pl
from jax.experimental.pallas import tpu as pltpu
from jax.experimental.pallas import tpu_sc as plsc

N_ORDER = 10000
N_RIDER = 10000
E = 160000
D_ORDER = 256
D_RIDER = 128
D_EDGE = 16
H = 256
HH = 128
NT = 16
EPT = E // NT
CH = 80
NCH = EPT // CH
SCC = 5
CPS = NCH // SCC
NP = 10240
RB = 10
ROWS = N_ORDER // RB


def _order_dense_body(x_ref, ws_ref, wp_ref, att_ref, bp_ref,
                      hlo_ref, hhi_ref, plo_ref, phi_ref, a_ref, amax_ref):
    i = pl.program_id(0)
    x = x_ref[...]
    h = jnp.dot(x, ws_ref[...], preferred_element_type=jnp.float32)
    hlo_ref[...] = h[:, :HH]
    hhi_ref[...] = h[:, HH:]
    a = jnp.dot(h, att_ref[...], preferred_element_type=jnp.float32)
    a_ref[...] = a
    proj = jnp.dot(x, wp_ref[...], preferred_element_type=jnp.float32)
    proj = proj + bp_ref[...]
    plo_ref[...] = proj[:, :HH]
    phi_ref[...] = proj[:, HH:]
    m = jnp.max(a).reshape(1, 1)

    @pl.when(i == 0)
    def _():
        amax_ref[...] = m

    @pl.when(i != 0)
    def _():
        amax_ref[...] = jnp.maximum(amax_ref[...], m)


def _order_dense(x_order, W_src, W_proj, att_src, b_proj):
    att2 = att_src.reshape(H, 1)
    bp2 = b_proj.reshape(1, H)
    out_shapes = [
        jax.ShapeDtypeStruct((N_ORDER, HH), jnp.float32),
        jax.ShapeDtypeStruct((N_ORDER, HH), jnp.float32),
        jax.ShapeDtypeStruct((N_ORDER, HH), jnp.float32),
        jax.ShapeDtypeStruct((N_ORDER, HH), jnp.float32),
        jax.ShapeDtypeStruct((N_ORDER, 1), jnp.float32),
        jax.ShapeDtypeStruct((1, 1), jnp.float32),
    ]
    row_spec = pl.BlockSpec((ROWS, HH), lambda i: (i, 0))
    return pl.pallas_call(
        _order_dense_body,
        grid=(RB,),
        in_specs=[
            pl.BlockSpec((ROWS, D_ORDER), lambda i: (i, 0)),
            pl.BlockSpec((D_ORDER, H), lambda i: (0, 0)),
            pl.BlockSpec((D_ORDER, H), lambda i: (0, 0)),
            pl.BlockSpec((H, 1), lambda i: (0, 0)),
            pl.BlockSpec((1, H), lambda i: (0, 0)),
        ],
        out_specs=[
            row_spec, row_spec, row_spec, row_spec,
            pl.BlockSpec((ROWS, 1), lambda i: (i, 0)),
            pl.BlockSpec((1, 1), lambda i: (0, 0)),
        ],
        out_shape=out_shapes,
    )(x_order, W_src, W_proj, att2, bp2)


def _rider_dense_body(x_ref, wd_ref, att_ref, a_ref, amax_ref):
    i = pl.program_id(0)
    hd = jnp.dot(x_ref[...], wd_ref[...], preferred_element_type=jnp.float32)
    a = jnp.dot(hd, att_ref[...], preferred_element_type=jnp.float32)
    a_ref[...] = a
    m = jnp.max(a).reshape(1, 1)

    @pl.when(i == 0)
    def _():
        amax_ref[...] = m

    @pl.when(i != 0)
    def _():
        amax_ref[...] = jnp.maximum(amax_ref[...], m)


def _rider_dense(x_rider, W_dst, att_dst):
    att2 = att_dst.reshape(H, 1)
    return pl.pallas_call(
        _rider_dense_body,
        grid=(RB,),
        in_specs=[
            pl.BlockSpec((ROWS, D_RIDER), lambda i: (i, 0)),
            pl.BlockSpec((D_RIDER, H), lambda i: (0, 0)),
            pl.BlockSpec((H, 1), lambda i: (0, 0)),
        ],
        out_specs=[
            pl.BlockSpec((ROWS, 1), lambda i: (i, 0)),
            pl.BlockSpec((1, 1), lambda i: (0, 0)),
        ],
        out_shape=[
            jax.ShapeDtypeStruct((N_RIDER, 1), jnp.float32),
            jax.ShapeDtypeStruct((1, 1), jnp.float32),
        ],
    )(x_rider, W_dst, att2)


def _gate_body(a_ref, w1_ref, b1_ref, w2_ref, b2_ref, o_ref):
    g = jnp.dot(a_ref[...], w1_ref[...], preferred_element_type=jnp.float32)
    g = jnp.maximum(g + b1_ref[...], 0.0)
    s = jnp.dot(g, w2_ref[...], preferred_element_type=jnp.float32)
    o_ref[...] = jax.nn.sigmoid(s + b2_ref[...])


def _gate(edge_attr, Wg1, bg1, Wg2, bg2):
    A = edge_attr.reshape(E // 8, 128)
    eye8 = jnp.eye(8, dtype=jnp.float32)
    W1b = jnp.kron(eye8, Wg1)
    b1b = jnp.tile(bg1, 8).reshape(1, 128)
    W2b = jnp.kron(eye8, Wg2)
    b2b = jnp.tile(bg2, 8).reshape(1, 8)
    rows = (E // 8) // RB
    out = pl.pallas_call(
        _gate_body,
        grid=(RB,),
        in_specs=[
            pl.BlockSpec((rows, 128), lambda i: (i, 0)),
            pl.BlockSpec((128, 128), lambda i: (0, 0)),
            pl.BlockSpec((1, 128), lambda i: (0, 0)),
            pl.BlockSpec((128, 8), lambda i: (0, 0)),
            pl.BlockSpec((1, 8), lambda i: (0, 0)),
        ],
        out_specs=pl.BlockSpec((rows, 8), lambda i: (i, 0)),
        out_shape=jax.ShapeDtypeStruct((E // 8, 8), jnp.float32),
    )(A, W1b, b1b, W2b, b2b)
    return out.reshape(E)


def _combine_body(p0_ref, p1_ref, g_ref, o_ref):
    o_ref[...] = (p0_ref[...] + p1_ref[...]) * g_ref[...]


def _combine(p0, p1, gate):
    shp = (E // 128, 128)
    return pl.pallas_call(
        _combine_body,
        out_shape=jax.ShapeDtypeStruct(shp, jnp.float32),
    )(p0.reshape(shp), p1.reshape(shp), gate.reshape(shp)).reshape(E)


def _rload(ref, r, j):
    return ref[r, pl.ds(16 * j, 16)]


def _rstore(ref, r, j, v):
    ref[r, pl.ds(16 * j, 16)] = v


def _dsplit(d16):
    return [lax.shift_right_logical(d16, 7), jnp.bitwise_and(d16, 127)]


@functools.partial(
    pl.kernel,
    out_type=[
        jax.ShapeDtypeStruct((2, NT, SCC, CPS, CH), jnp.float32),
        jax.ShapeDtypeStruct((2, NT, SCC, CPS, CH), jnp.float32),
    ],
    mesh=plsc.VectorSubcoreMesh(core_axis_name="c", subcore_axis_name="s"),
    compiler_params=pltpu.CompilerParams(
        use_tc_tiling_on_sc=False, needs_layout_passes=False),
    scratch_types=[
        pltpu.VMEM((CPS, CH), jnp.int32),
        pltpu.VMEM((CPS, CH), jnp.int32),
        pltpu.VMEM((CPS, CH), jnp.float32),
        pltpu.VMEM((80, 128), jnp.float32),
        pltpu.VMEM((80, 128), jnp.float32),
        pltpu.VMEM((80, 128), jnp.float32),
        pltpu.VMEM((80, 128), jnp.float32),
        pltpu.VMEM((16,), jnp.float32),
        pltpu.VMEM((1, CH), jnp.int32),
        pltpu.SemaphoreType.DMA,
        pltpu.SemaphoreType.DMA,
        pltpu.SemaphoreType.DMA,
        pltpu.SemaphoreType.DMA,
        pltpu.VMEM_SHARED((N_RIDER, HH), jnp.float32),
        pltpu.VMEM_SHARED((80, 128), jnp.float32),
    ],
)
def _sc_edges(src4_h, dst4_h, asrc_h, adst_h, cvec_h, zrows_h, brows_h,
              hlo_h, hhi_h, plo_h, phi_h, out_h, exb_h,
              srcsc_v, dstsc_v, stagef_v, den_v, rowsA, rowsB, rowsC,
              cvec_v, idxr_v, sem0, sem1, sem2, sem3, rider_s, dfin_s):
    c = lax.axis_index("c")
    t = lax.axis_index("s")

    pltpu.sync_copy(asrc_h, rowsB)
    pltpu.sync_copy(adst_h, rowsA)
    pltpu.sync_copy(cvec_h, cvec_v)
    pltpu.sync_copy(zrows_h, den_v)
    pltpu.sync_copy(brows_h.at[c],
                    rider_s.at[pl.ds((N_RIDER // NT) * t, N_RIDER // NT)])

    @pl.when(t == 0)
    def _():
        pltpu.sync_copy(zrows_h, dfin_s)

    for u in range(CH // 16):
        idxr_v[0, pl.ds(16 * u, 16)] = lax.iota(jnp.int32, 16) + 16 * u

    cv = cvec_v[...]

    def superA(s, carry):
        pltpu.sync_copy(src4_h.at[t, s], srcsc_v)
        pltpu.sync_copy(dst4_h.at[t, s], dstsc_v)

        def rA(r, cc):
            for u in range(CH // 16):
                s16 = _rload(srcsc_v, r, u)
                d16 = _rload(dstsc_v, r, u)
                av = plsc.load_gather(rowsB, _dsplit(s16))
                bv = plsc.load_gather(rowsA, _dsplit(d16))
                x = av + bv
                ex = jnp.exp(jnp.where(x >= 0.0, x, 0.2 * x) - cv)
                _rstore(stagef_v, r, u, ex)
                plsc.addupdate_scatter(den_v, _dsplit(d16), ex)
            return cc

        lax.fori_loop(0, CPS, rA, 0)
        pltpu.sync_copy(stagef_v, exb_h.at[c, t, s])
        return carry

    lax.fori_loop(0, SCC, superA, 0)

    plsc.subcore_barrier()
    pltpu.sync_copy(den_v, dfin_s.at[idxr_v.at[0]], add=True)
    plsc.subcore_barrier()
    pltpu.sync_copy(dfin_s, den_v)

    il = lax.iota(jnp.int32, 16)

    def phase23(h_h, p_h):
        def scale_scatter(r, gbuf):
            def sk(k, cc):
                d16 = dstsc_v[r, pl.ds(16 * k, 16)]
                dv = plsc.load_gather(den_v, _dsplit(d16))
                a16 = stagef_v[r, pl.ds(16 * k, 16)] / (dv + 1e-16)
                for rr in range(16):
                    ab = jnp.full((16,), a16[rr], jnp.float32)
                    row = 16 * k + rr
                    for j in range(HH // 16):
                        gbuf[row, pl.ds(16 * j, 16)] = (
                            gbuf[row, pl.ds(16 * j, 16)] * ab)
                return cc

            lax.fori_loop(0, CH // 16, sk, 0)
            pltpu.sync_copy(gbuf, rider_s.at[dstsc_v.at[r]], add=True)

        def superB(s, carry):
            pltpu.sync_copy(src4_h.at[t, s], srcsc_v)
            pltpu.sync_copy(dst4_h.at[t, s], dstsc_v)
            pltpu.sync_copy(exb_h.at[c, t, s], stagef_v)
            pltpu.make_async_copy(h_h.at[srcsc_v.at[0]], rowsB, sem0).start()

            def rB(r, cc):
                nxt = r + 1

                @pl.when(jnp.logical_and(nxt < CPS, nxt % 2 == 1))
                def _():
                    pltpu.make_async_copy(
                        h_h.at[srcsc_v.at[nxt]], rowsC, sem1).start()

                @pl.when(jnp.logical_and(nxt < CPS, nxt % 2 == 0))
                def _():
                    pltpu.make_async_copy(
                        h_h.at[srcsc_v.at[nxt]], rowsB, sem0).start()

                @pl.when(r % 2 == 0)
                def _():
                    pltpu.make_async_copy(
                        h_h.at[srcsc_v.at[r]], rowsB, sem0).wait()
                    scale_scatter(r, rowsB)

                @pl.when(r % 2 == 1)
                def _():
                    pltpu.make_async_copy(
                        h_h.at[srcsc_v.at[r]], rowsC, sem1).wait()
                    scale_scatter(r, rowsC)

                return cc

            lax.fori_loop(0, CPS, rB, 0)
            return carry

        lax.fori_loop(0, SCC, superB, 0)
        plsc.subcore_barrier()

        def dot_rows(r, pbuf, rbuf):
            def dk(k, cc):
                def drr(rr, res):
                    row = 16 * k + rr
                    acc = (pbuf[row, pl.ds(0, 16)] *
                           rbuf[row, pl.ds(0, 16)])
                    for j in range(1, HH // 16):
                        acc = acc + (pbuf[row, pl.ds(16 * j, 16)] *
                                     rbuf[row, pl.ds(16 * j, 16)])
                    return jnp.where(il == rr, jnp.sum(acc), res)

                res = lax.fori_loop(0, 16, drr,
                                    jnp.zeros((16,), jnp.float32))
                stagef_v[r, pl.ds(16 * k, 16)] = res
                return cc

            lax.fori_loop(0, CH // 16, dk, 0)

        def superC(s, carry):
            pltpu.sync_copy(src4_h.at[t, s], srcsc_v)
            pltpu.sync_copy(dst4_h.at[t, s], dstsc_v)
            pltpu.make_async_copy(p_h.at[srcsc_v.at[0]], rowsB, sem0).start()
            pltpu.make_async_copy(
                rider_s.at[dstsc_v.at[0]], rowsA, sem2).start()

            def rC(r, cc):
                nxt = r + 1

                @pl.when(jnp.logical_and(nxt < CPS, nxt % 2 == 1))
                def _():
                    pltpu.make_async_copy(
                        p_h.at[srcsc_v.at[nxt]], rowsC, sem1).start()
                    pltpu.make_async_copy(
                        rider_s.at[dstsc_v.at[nxt]], den_v, sem3).start()

                @pl.when(jnp.logical_and(nxt < CPS, nxt % 2 == 0))
                def _():
                    pltpu.make_async_copy(
                        p_h.at[srcsc_v.at[nxt]], rowsB, sem0).start()
                    pltpu.make_async_copy(
                        rider_s.at[dstsc_v.at[nxt]], rowsA, sem2).start()

                @pl.when(r % 2 == 0)
                def _():
                    pltpu.make_async_copy(
                        p_h.at[srcsc_v.at[r]], rowsB, sem0).wait()
                    pltpu.make_async_copy(
                        rider_s.at[dstsc_v.at[r]], rowsA, sem2).wait()
                    dot_rows(r, rowsB, rowsA)

                @pl.when(r % 2 == 1)
                def _():
                    pltpu.make_async_copy(
                        p_h.at[srcsc_v.at[r]], rowsC, sem1).wait()
                    pltpu.make_async_copy(
                        rider_s.at[dstsc_v.at[r]], den_v, sem3).wait()
                    dot_rows(r, rowsC, den_v)

                return cc

            lax.fori_loop(0, CPS, rC, 0)
            pltpu.sync_copy(stagef_v, out_h.at[c, t, s])
            return carry

        lax.fori_loop(0, SCC, superC, 0)

    @pl.when(c == 0)
    def _():
        phase23(hlo_h, plo_h)

    @pl.when(c == 1)
    def _():
        phase23(hhi_h, phi_h)


def kernel(x_order, x_rider, edge_index, edge_attr, W_src, W_dst,
           att_src, att_dst, bias_gat, W_proj, b_proj, Wg1, bg1, Wg2, bg2):
    src = edge_index[0].astype(jnp.int32)
    dst = edge_index[1].astype(jnp.int32)

    h_lo, h_hi, p_lo, p_hi, a_src, amax_s = _order_dense(
        x_order, W_src, W_proj, att_src, b_proj)
    a_dst, amax_d = _rider_dense(x_rider, W_dst, att_dst)
    gate = _gate(edge_attr, Wg1, bg1, Wg2, bg2)

    c = jnp.maximum(amax_s[0, 0] + amax_d[0, 0], 0.0)
    cvec = jnp.broadcast_to(c, (16,))
    pad = jnp.zeros((NP - N_ORDER,), jnp.float32)
    asrc2d = jnp.concatenate([a_src.reshape(N_ORDER), pad]).reshape(80, 128)
    adst2d = jnp.concatenate([a_dst.reshape(N_RIDER), pad]).reshape(80, 128)
    zrows = jnp.zeros((80, 128), jnp.float32)
    brows = jnp.broadcast_to(bias_gat.reshape(2, 1, HH),
                             (2, N_RIDER // NT, HH))

    partial, _ = _sc_edges(
        src.reshape(NT, SCC, CPS, CH), dst.reshape(NT, SCC, CPS, CH),
        asrc2d, adst2d, cvec, zrows, brows, h_lo, h_hi, p_lo, p_hi)

    p = partial.reshape(2, E)
    return _combine(p[0], p[1], gate)


# speedup vs baseline: 7.4170x; 1.0013x over previous
import functools

import jax
import jax.numpy as jnp
from jax import lax
from jax.experimental import pallas as pl
from jax.experimental.pallas import tpu as pltpu
from jax.experimental.pallas import tpu_sc as plsc

N_ORDER = 10000
N_RIDER = 10000
E = 160000
D_ORDER = 256
D_RIDER = 128
D_EDGE = 16
H = 256
HH = 128
NT = 16
EPT = E // NT
CH = 80
NCH = EPT // CH
SCC = 5
CPS = NCH // SCC
NP = 10240
RB = 10
ROWS = N_ORDER // RB


def _order_dense_body(x_ref, ws_ref, wp_ref, att_ref, bp_ref,
                      hlo_ref, hhi_ref, plo_ref, phi_ref, a_ref, amax_ref):
    i = pl.program_id(0)
    x = x_ref[...]
    h = jnp.dot(x, ws_ref[...], preferred_element_type=jnp.float32)
    hlo_ref[...] = h[:, :HH]
    hhi_ref[...] = h[:, HH:]
    a = jnp.dot(h, att_ref[...], preferred_element_type=jnp.float32)
    a_ref[...] = a
    proj = jnp.dot(x, wp_ref[...], preferred_element_type=jnp.float32)
    proj = proj + bp_ref[...]
    plo_ref[...] = proj[:, :HH]
    phi_ref[...] = proj[:, HH:]
    m = jnp.max(a).reshape(1, 1)

    @pl.when(i == 0)
    def _():
        amax_ref[...] = m

    @pl.when(i != 0)
    def _():
        amax_ref[...] = jnp.maximum(amax_ref[...], m)


def _order_dense(x_order, W_src, W_proj, att_src, b_proj):
    att2 = att_src.reshape(H, 1)
    bp2 = b_proj.reshape(1, H)
    out_shapes = [
        jax.ShapeDtypeStruct((N_ORDER, HH), jnp.float32),
        jax.ShapeDtypeStruct((N_ORDER, HH), jnp.float32),
        jax.ShapeDtypeStruct((N_ORDER, HH), jnp.float32),
        jax.ShapeDtypeStruct((N_ORDER, HH), jnp.float32),
        jax.ShapeDtypeStruct((N_ORDER, 1), jnp.float32),
        jax.ShapeDtypeStruct((1, 1), jnp.float32),
    ]
    row_spec = pl.BlockSpec((ROWS, HH), lambda i: (i, 0))
    return pl.pallas_call(
        _order_dense_body,
        grid=(RB,),
        in_specs=[
            pl.BlockSpec((ROWS, D_ORDER), lambda i: (i, 0)),
            pl.BlockSpec((D_ORDER, H), lambda i: (0, 0)),
            pl.BlockSpec((D_ORDER, H), lambda i: (0, 0)),
            pl.BlockSpec((H, 1), lambda i: (0, 0)),
            pl.BlockSpec((1, H), lambda i: (0, 0)),
        ],
        out_specs=[
            row_spec, row_spec, row_spec, row_spec,
            pl.BlockSpec((ROWS, 1), lambda i: (i, 0)),
            pl.BlockSpec((1, 1), lambda i: (0, 0)),
        ],
        out_shape=out_shapes,
    )(x_order, W_src, W_proj, att2, bp2)


def _rider_dense_body(x_ref, wd_ref, att_ref, a_ref, amax_ref):
    i = pl.program_id(0)
    hd = jnp.dot(x_ref[...], wd_ref[...], preferred_element_type=jnp.float32)
    a = jnp.dot(hd, att_ref[...], preferred_element_type=jnp.float32)
    a_ref[...] = a
    m = jnp.max(a).reshape(1, 1)

    @pl.when(i == 0)
    def _():
        amax_ref[...] = m

    @pl.when(i != 0)
    def _():
        amax_ref[...] = jnp.maximum(amax_ref[...], m)


def _rider_dense(x_rider, W_dst, att_dst):
    att2 = att_dst.reshape(H, 1)
    return pl.pallas_call(
        _rider_dense_body,
        grid=(RB,),
        in_specs=[
            pl.BlockSpec((ROWS, D_RIDER), lambda i: (i, 0)),
            pl.BlockSpec((D_RIDER, H), lambda i: (0, 0)),
            pl.BlockSpec((H, 1), lambda i: (0, 0)),
        ],
        out_specs=[
            pl.BlockSpec((ROWS, 1), lambda i: (i, 0)),
            pl.BlockSpec((1, 1), lambda i: (0, 0)),
        ],
        out_shape=[
            jax.ShapeDtypeStruct((N_RIDER, 1), jnp.float32),
            jax.ShapeDtypeStruct((1, 1), jnp.float32),
        ],
    )(x_rider, W_dst, att2)


def _gate_body(a_ref, w1_ref, b1_ref, w2_ref, b2_ref, o_ref):
    g = jnp.dot(a_ref[...], w1_ref[...], preferred_element_type=jnp.float32)
    g = jnp.maximum(g + b1_ref[...], 0.0)
    s = jnp.dot(g, w2_ref[...], preferred_element_type=jnp.float32)
    o_ref[...] = jax.nn.sigmoid(s + b2_ref[...])


def _gate(edge_attr, Wg1, bg1, Wg2, bg2):
    A = edge_attr.reshape(E // 8, 128)
    eye8 = jnp.eye(8, dtype=jnp.float32)
    W1b = jnp.kron(eye8, Wg1)
    b1b = jnp.tile(bg1, 8).reshape(1, 128)
    W2b = jnp.kron(eye8, Wg2)
    b2b = jnp.tile(bg2, 8).reshape(1, 8)
    rows = (E // 8) // RB
    out = pl.pallas_call(
        _gate_body,
        grid=(RB,),
        in_specs=[
            pl.BlockSpec((rows, 128), lambda i: (i, 0)),
            pl.BlockSpec((128, 128), lambda i: (0, 0)),
            pl.BlockSpec((1, 128), lambda i: (0, 0)),
            pl.BlockSpec((128, 8), lambda i: (0, 0)),
            pl.BlockSpec((1, 8), lambda i: (0, 0)),
        ],
        out_specs=pl.BlockSpec((rows, 8), lambda i: (i, 0)),
        out_shape=jax.ShapeDtypeStruct((E // 8, 8), jnp.float32),
    )(A, W1b, b1b, W2b, b2b)
    return out.reshape(E)


def _combine_body(p0_ref, p1_ref, g_ref, o_ref):
    o_ref[...] = (p0_ref[...] + p1_ref[...]) * g_ref[...]


def _combine(p0, p1, gate):
    shp = (E // 128, 128)
    return pl.pallas_call(
        _combine_body,
        out_shape=jax.ShapeDtypeStruct(shp, jnp.float32),
    )(p0.reshape(shp), p1.reshape(shp), gate.reshape(shp)).reshape(E)


def _rload(ref, r, j):
    return ref[r, pl.ds(16 * j, 16)]


def _rstore(ref, r, j, v):
    ref[r, pl.ds(16 * j, 16)] = v


def _dsplit(d16):
    return [lax.shift_right_logical(d16, 7), jnp.bitwise_and(d16, 127)]


@functools.partial(
    pl.kernel,
    out_type=[
        jax.ShapeDtypeStruct((2, NT, SCC, CPS, CH), jnp.float32),
        jax.ShapeDtypeStruct((2, NT, SCC, CPS, CH), jnp.float32),
    ],
    mesh=plsc.VectorSubcoreMesh(core_axis_name="c", subcore_axis_name="s"),
    compiler_params=pltpu.CompilerParams(
        use_tc_tiling_on_sc=False, needs_layout_passes=False),
    scratch_types=[
        pltpu.VMEM((CPS, CH), jnp.int32),
        pltpu.VMEM((CPS, CH), jnp.int32),
        pltpu.VMEM((CPS, CH), jnp.float32),
        pltpu.VMEM((80, 128), jnp.float32),
        pltpu.VMEM((80, 128), jnp.float32),
        pltpu.VMEM((80, 128), jnp.float32),
        pltpu.VMEM((80, 128), jnp.float32),
        pltpu.VMEM((16,), jnp.float32),
        pltpu.VMEM((1, CH), jnp.int32),
        pltpu.SemaphoreType.DMA,
        pltpu.SemaphoreType.DMA,
        pltpu.SemaphoreType.DMA,
        pltpu.SemaphoreType.DMA,
        pltpu.VMEM_SHARED((N_RIDER, HH), jnp.float32),
        pltpu.VMEM_SHARED((80, 128), jnp.float32),
    ],
)
def _sc_edges(src4_h, dst4_h, asrc_h, adst_h, cvec_h, zrows_h, brows_h,
              hlo_h, hhi_h, plo_h, phi_h, out_h, exb_h,
              srcsc_v, dstsc_v, stagef_v, den_v, rowsA, rowsB, rowsC,
              cvec_v, idxr_v, sem0, sem1, sem2, sem3, rider_s, dfin_s):
    c = lax.axis_index("c")
    t = lax.axis_index("s")

    pltpu.sync_copy(asrc_h, rowsB)
    pltpu.sync_copy(adst_h, rowsA)
    pltpu.sync_copy(cvec_h, cvec_v)
    pltpu.sync_copy(zrows_h, den_v)
    pltpu.sync_copy(brows_h.at[c],
                    rider_s.at[pl.ds((N_RIDER // NT) * t, N_RIDER // NT)])

    @pl.when(t == 0)
    def _():
        pltpu.sync_copy(zrows_h, dfin_s)

    for u in range(CH // 16):
        idxr_v[0, pl.ds(16 * u, 16)] = lax.iota(jnp.int32, 16) + 16 * u

    cv = cvec_v[...]

    def superA(s, carry):
        pltpu.sync_copy(src4_h.at[t, s], srcsc_v)
        pltpu.sync_copy(dst4_h.at[t, s], dstsc_v)

        def rA(r, cc):
            for u in range(CH // 16):
                s16 = _rload(srcsc_v, r, u)
                d16 = _rload(dstsc_v, r, u)
                av = plsc.load_gather(rowsB, _dsplit(s16))
                bv = plsc.load_gather(rowsA, _dsplit(d16))
                x = av + bv
                ex = jnp.exp(jnp.where(x >= 0.0, x, 0.2 * x) - cv)
                _rstore(stagef_v, r, u, ex)
                plsc.addupdate_scatter(den_v, _dsplit(d16), ex)
            return cc

        lax.fori_loop(0, CPS, rA, 0)
        pltpu.sync_copy(stagef_v, exb_h.at[c, t, s])
        return carry

    lax.fori_loop(0, SCC, superA, 0)

    plsc.subcore_barrier()
    pltpu.sync_copy(den_v, dfin_s.at[idxr_v.at[0]], add=True)
    plsc.subcore_barrier()
    pltpu.sync_copy(dfin_s, den_v)

    il = lax.iota(jnp.int32, 16)

    def phase23(h_h, p_h):
        def scale_scatter(r, gbuf):
            def sk(k, cc):
                d16 = dstsc_v[r, pl.ds(16 * k, 16)]
                dv = plsc.load_gather(den_v, _dsplit(d16))
                a16 = stagef_v[r, pl.ds(16 * k, 16)] / (dv + 1e-16)
                for rr in range(16):
                    ab = jnp.full((16,), a16[rr], jnp.float32)
                    row = 16 * k + rr
                    for j in range(HH // 16):
                        gbuf[row, pl.ds(16 * j, 16)] = (
                            gbuf[row, pl.ds(16 * j, 16)] * ab)
                return cc

            lax.fori_loop(0, CH // 16, sk, 0)

        def superB(s, carry):
            pltpu.sync_copy(src4_h.at[t, s], srcsc_v)
            pltpu.sync_copy(dst4_h.at[t, s], dstsc_v)
            pltpu.sync_copy(exb_h.at[c, t, s], stagef_v)
            pltpu.make_async_copy(h_h.at[srcsc_v.at[0]], rowsB, sem0).start()

            def rB(r, cc):
                nxt = r + 1
                live = nxt < CPS

                @pl.when(jnp.logical_and(
                    live, jnp.logical_and(nxt % 2 == 1, r >= 1)))
                def _():
                    pltpu.make_async_copy(
                        rowsC, rider_s.at[dstsc_v.at[r - 1]], sem3).wait()

                @pl.when(jnp.logical_and(
                    live, jnp.logical_and(nxt % 2 == 0, r >= 1)))
                def _():
                    pltpu.make_async_copy(
                        rowsB, rider_s.at[dstsc_v.at[r - 1]], sem2).wait()

                @pl.when(jnp.logical_and(live, nxt % 2 == 1))
                def _():
                    pltpu.make_async_copy(
                        h_h.at[srcsc_v.at[nxt]], rowsC, sem1).start()

                @pl.when(jnp.logical_and(live, nxt % 2 == 0))
                def _():
                    pltpu.make_async_copy(
                        h_h.at[srcsc_v.at[nxt]], rowsB, sem0).start()

                @pl.when(r % 2 == 0)
                def _():
                    pltpu.make_async_copy(
                        h_h.at[srcsc_v.at[r]], rowsB, sem0).wait()
                    scale_scatter(r, rowsB)
                    pltpu.make_async_copy(
                        rowsB, rider_s.at[dstsc_v.at[r]], sem2,
                    ).start(add=True)

                @pl.when(r % 2 == 1)
                def _():
                    pltpu.make_async_copy(
                        h_h.at[srcsc_v.at[r]], rowsC, sem1).wait()
                    scale_scatter(r, rowsC)
                    pltpu.make_async_copy(
                        rowsC, rider_s.at[dstsc_v.at[r]], sem3,
                    ).start(add=True)

                return cc

            lax.fori_loop(0, CPS, rB, 0)
            pltpu.make_async_copy(
                rowsC, rider_s.at[dstsc_v.at[CPS - 2]], sem3).wait()
            pltpu.make_async_copy(
                rowsB, rider_s.at[dstsc_v.at[CPS - 1]], sem2).wait()
            return carry

        lax.fori_loop(0, SCC, superB, 0)
        plsc.subcore_barrier()

        def dot_rows(r, pbuf, rbuf):
            def dk(k, cc):
                def drr(rr, res):
                    row = 16 * k + rr
                    acc = (pbuf[row, pl.ds(0, 16)] *
                           rbuf[row, pl.ds(0, 16)])
                    for j in range(1, HH // 16):
                        acc = acc + (pbuf[row, pl.ds(16 * j, 16)] *
                                     rbuf[row, pl.ds(16 * j, 16)])
                    return jnp.where(il == rr, jnp.sum(acc), res)

                res = lax.fori_loop(0, 16, drr,
                                    jnp.zeros((16,), jnp.float32))
                stagef_v[r, pl.ds(16 * k, 16)] = res
                return cc

            lax.fori_loop(0, CH // 16, dk, 0)

        def superC(s, carry):
            pltpu.sync_copy(src4_h.at[t, s], srcsc_v)
            pltpu.sync_copy(dst4_h.at[t, s], dstsc_v)
            pltpu.make_async_copy(p_h.at[srcsc_v.at[0]], rowsB, sem0).start()
            pltpu.make_async_copy(
                rider_s.at[dstsc_v.at[0]], rowsA, sem2).start()

            def rC(r, cc):
                nxt = r + 1

                @pl.when(jnp.logical_and(nxt < CPS, nxt % 2 == 1))
                def _():
                    pltpu.make_async_copy(
                        p_h.at[srcsc_v.at[nxt]], rowsC, sem1).start()
                    pltpu.make_async_copy(
                        rider_s.at[dstsc_v.at[nxt]], den_v, sem3).start()

                @pl.when(jnp.logical_and(nxt < CPS, nxt % 2 == 0))
                def _():
                    pltpu.make_async_copy(
                        p_h.at[srcsc_v.at[nxt]], rowsB, sem0).start()
                    pltpu.make_async_copy(
                        rider_s.at[dstsc_v.at[nxt]], rowsA, sem2).start()

                @pl.when(r % 2 == 0)
                def _():
                    pltpu.make_async_copy(
                        p_h.at[srcsc_v.at[r]], rowsB, sem0).wait()
                    pltpu.make_async_copy(
                        rider_s.at[dstsc_v.at[r]], rowsA, sem2).wait()
                    dot_rows(r, rowsB, rowsA)

                @pl.when(r % 2 == 1)
                def _():
                    pltpu.make_async_copy(
                        p_h.at[srcsc_v.at[r]], rowsC, sem1).wait()
                    pltpu.make_async_copy(
                        rider_s.at[dstsc_v.at[r]], den_v, sem3).wait()
                    dot_rows(r, rowsC, den_v)

                return cc

            lax.fori_loop(0, CPS, rC, 0)
            pltpu.sync_copy(stagef_v, out_h.at[c, t, s])
            return carry

        lax.fori_loop(0, SCC, superC, 0)

    @pl.when(c == 0)
    def _():
        phase23(hlo_h, plo_h)

    @pl.when(c == 1)
    def _():
        phase23(hhi_h, phi_h)


def kernel(x_order, x_rider, edge_index, edge_attr, W_src, W_dst,
           att_src, att_dst, bias_gat, W_proj, b_proj, Wg1, bg1, Wg2, bg2):
    src = edge_index[0].astype(jnp.int32)
    dst = edge_index[1].astype(jnp.int32)

    h_lo, h_hi, p_lo, p_hi, a_src, amax_s = _order_dense(
        x_order, W_src, W_proj, att_src, b_proj)
    a_dst, amax_d = _rider_dense(x_rider, W_dst, att_dst)
    gate = _gate(edge_attr, Wg1, bg1, Wg2, bg2)

    c = jnp.maximum(amax_s[0, 0] + amax_d[0, 0], 0.0)
    cvec = jnp.broadcast_to(c, (16,))
    pad = jnp.zeros((NP - N_ORDER,), jnp.float32)
    asrc2d = jnp.concatenate([a_src.reshape(N_ORDER), pad]).reshape(80, 128)
    adst2d = jnp.concatenate([a_dst.reshape(N_RIDER), pad]).reshape(80, 128)
    zrows = jnp.zeros((80, 128), jnp.float32)
    brows = jnp.broadcast_to(bias_gat.reshape(2, 1, HH),
                             (2, N_RIDER // NT, HH))

    partial, _ = _sc_edges(
        src.reshape(NT, SCC, CPS, CH), dst.reshape(NT, SCC, CPS, CH),
        asrc2d, adst2d, cvec, zrows, brows, h_lo, h_hi, p_lo, p_hi)

    p = partial.reshape(2, E)
    return _combine(p[0], p[1], gate)
